# baseline (speedup 1.0000x reference)
import functools

import jax
import jax.numpy as jnp
from jax import lax
from jax.experimental import pallas as pl
from jax.experimental import layout as jlayout
from jax.experimental.pallas import tpu as pltpu
from jax.experimental.pallas import tpu_sc as plsc

D = 128
C = 128
NBUF = 5

_info = plsc.get_sparse_core_info()
_NC, _NS = _info.num_cores, _info.num_subcores
NW = _NC * _NS

def _body(x_hbm, table_hbm, out_hbm, idx2d, idx_flat, *scratch):
    wid = lax.axis_index("s") * _NC + lax.axis_index("c")
    hist = x_hbm.shape[1]
    rows_pw = x_hbm.shape[0] // NW
    n_idx = rows_pw * hist
    nchunk = n_idx // C
    base = wid * n_idx
    rows = scratch[:NBUF]
    gsem = scratch[NBUF:2 * NBUF]
    wsem = scratch[2 * NBUF:]

    pltpu.sync_copy(x_hbm.at[pl.ds(wid * rows_pw, rows_pw)], idx2d)

    starts = []
    c = 0
    while c + 16 < hist:
        starts.append(c)
        c += 16
    starts.append(hist - 16)

    @pl.loop(0, rows_pw, unroll=2)
    def _(r):
        for c0 in starts:
            idx_flat[pl.ds(r * hist + c0, 16)] = idx2d[r, pl.ds(c0, 16)]

    @pl.loop(0, nchunk, step=NBUF)
    def _(j0):
        for b in range(NBUF):
            j = j0 + b

            @pl.when(j0 >= NBUF)
            def _():
                pltpu.make_async_copy(
                    rows[b],
                    out_hbm.at[pl.ds(base + (j - NBUF) * C, C)],
                    wsem[b],
                ).wait()

            pltpu.async_copy(
                table_hbm.at[idx_flat.at[pl.ds(j * C, C)]], rows[b], gsem[b]
            )

        for b in range(NBUF):
            j = j0 + b
            pltpu.make_async_copy(
                table_hbm.at[idx_flat.at[pl.ds(j * C, C)]], rows[b], gsem[b]
            ).wait()
            pltpu.async_copy(
                rows[b], out_hbm.at[pl.ds(base + j * C, C)], wsem[b]
            )

    for b in range(NBUF):
        j = nchunk - NBUF + b
        pltpu.make_async_copy(
            rows[b], out_hbm.at[pl.ds(base + j * C, C)], wsem[b]
        ).wait()


@functools.cache
def _jitted():
    fmt = jlayout.Format(
        jlayout.Layout(major_to_minor=(0, 1, 2), tiling=()),
        jax.sharding.SingleDeviceSharding(jax.devices()[0]),
    )
    return jax.jit(_kernel_impl, out_shardings=fmt)


def kernel(x, table):
    return _jitted()(x, table)


def _kernel_impl(x, table):
    batch, hist = x.shape
    total = batch * hist
    rows_pw = batch // NW

    mesh = plsc.VectorSubcoreMesh(core_axis_name="c", subcore_axis_name="s")
    run = pl.kernel(
        _body,
        out_type=jax.ShapeDtypeStruct((total, D), jnp.float32),
        mesh=mesh,
        scratch_types=(
            [
                pltpu.VMEM((rows_pw, hist), jnp.int32),
                pltpu.VMEM((rows_pw * hist,), jnp.int32),
            ]
            + [pltpu.VMEM((C, D), jnp.float32)] * NBUF
            + [pltpu.SemaphoreType.DMA] * (2 * NBUF)
        ),
    )
    out = run(x.astype(jnp.int32), table)
    return out.reshape(batch, hist, D)
